# baseline (speedup 1.0000x reference)
import functools
import jax
import jax.numpy as jnp
from jax import lax
from jax.experimental import pallas as pl
from jax.experimental.pallas import tpu as pltpu
from jax.experimental.pallas import tpu_sc as plsc

N_RAYS = 16384
S = 128
C_OUT = 25
LATENT = 16

RB_A = 256
RB_C = 128
SC_WORKERS = 32
SC_CHUNK = N_RAYS // SC_WORKERS
SC_SUB = 256
GROUPS = SC_SUB // 16
ILV = 4


def _tk_body(rays_t_ref, radii_ref, t_ref, k_ref):
    rt = rays_t_ref[...]
    o = rt[0:3, :]
    d = rt[3:6, :]
    od = jnp.sum(o * d, axis=0, keepdims=True)
    dd = jnp.sum(d * d, axis=0, keepdims=True)
    oo = jnp.sum(o * o, axis=0, keepdims=True)
    r = radii_ref[...]
    disc = od * od - dd * (oo - r * r)
    disc = jnp.maximum(disc, 0.0)
    t = (-od + jnp.sqrt(disc)) / dd
    t_ref[...] = t
    k = jnp.sum((t > 0.0).astype(jnp.int32), axis=0)
    k_ref[...] = k.reshape(1, 1, RB_A)


def _compute_tk(rays_t, radii):
    grid = N_RAYS // RB_A
    return pl.pallas_call(
        _tk_body,
        grid=(grid,),
        in_specs=[
            pl.BlockSpec((8, RB_A), lambda i: (0, i)),
            pl.BlockSpec((S, 1), lambda i: (0, 0)),
        ],
        out_specs=[
            pl.BlockSpec((S, RB_A), lambda i: (0, i)),
            pl.BlockSpec((1, 1, RB_A), lambda i: (i, 0, 0)),
        ],
        out_shape=[
            jax.ShapeDtypeStruct((S, N_RAYS), jnp.float32),
            jax.ShapeDtypeStruct((grid, 1, RB_A), jnp.int32),
        ],
    )(rays_t, radii.reshape(S, 1))


def _codes_body(ct_ref, out_ref):
    ct = ct_ref[...]
    nrm = jnp.sqrt(jnp.sum(ct * ct, axis=0, keepdims=True))
    out_ref[...] = ct / jnp.maximum(nrm, 1e-12)


def _compute_codes(codes_t):
    return pl.pallas_call(
        _codes_body,
        out_shape=jax.ShapeDtypeStruct((LATENT, S), jnp.float32),
    )(codes_t)


def _sc_merge_body(t_hbm, k_hbm, out_hbm, t_v, o_v, k_v):
    wid = lax.axis_index("s") * 2 + lax.axis_index("c")
    big = jnp.float32(3e38)
    lanes = jnp.arange(16, dtype=jnp.int32)
    for sub in range(SC_CHUNK // SC_SUB):
        base = wid * SC_CHUNK + sub * SC_SUB
        pltpu.sync_copy(t_hbm.at[:, pl.ds(base, SC_SUB)], t_v)
        pltpu.sync_copy(k_hbm.at[pl.ds(base, SC_SUB)], k_v)
        for gb in range(GROUPS // ILV):
            rids = []
            los = []
            his = []
            for u in range(ILV):
                g = gb * ILV + u
                rids.append(g * 16 + lanes)
                kv = k_v[pl.ds(g * 16, 16)]
                los.append(kv - 1)
                his.append(kv)

            def step(j, carry):
                lohi = list(carry)
                jcol = jnp.full((16,), 0, jnp.int32) + j
                out = []
                for u in range(ILV):
                    lo, hi = lohi[2 * u], lohi[2 * u + 1]
                    vlo = lo >= 0
                    vhi = hi < S
                    ilo = jnp.clip(lo, 0, S - 1)
                    ihi = jnp.clip(hi, 0, S - 1)
                    tl = plsc.load_gather(t_v, [ilo, rids[u]])
                    th = plsc.load_gather(t_v, [ihi, rids[u]])
                    al = jnp.where(vlo, jnp.abs(tl), big)
                    ah = jnp.where(vhi, jnp.abs(th), big)
                    take = al <= ah
                    chosen = jnp.where(take, tl, th)
                    plsc.store_scatter(o_v, [jcol, rids[u]], chosen)
                    out.append(jnp.where(take, lo - 1, lo))
                    out.append(jnp.where(take, hi, hi + 1))
                return tuple(out)

            init = []
            for u in range(ILV):
                init.append(los[u])
                init.append(his[u])
            lax.fori_loop(0, S, step, tuple(init))
        pltpu.sync_copy(o_v, out_hbm.at[:, pl.ds(base, SC_SUB)])


def _sc_sort(t, k):
    mesh = plsc.VectorSubcoreMesh(core_axis_name="c", subcore_axis_name="s")
    fn = functools.partial(
        pl.kernel,
        mesh=mesh,
        out_type=jax.ShapeDtypeStruct((S, N_RAYS), jnp.float32),
        scratch_types=[
            pltpu.VMEM((S, SC_SUB), jnp.float32),
            pltpu.VMEM((S, SC_SUB), jnp.float32),
            pltpu.VMEM((SC_SUB,), jnp.int32),
        ],
        compiler_params=pltpu.CompilerParams(needs_layout_passes=False),
    )(_sc_merge_body)
    return fn(t, k)


RB_CODES = 1024


def _codes_planes_body(ct_ref, out_ref):
    p = pl.program_id(0)
    row = ct_ref[pl.ds(p, 1), :]
    out_ref[...] = jnp.broadcast_to(row[None], (1, RB_CODES, S))


def _codes_planes(codes_nt):
    return pl.pallas_call(
        _codes_planes_body,
        grid=(LATENT, N_RAYS // RB_CODES),
        in_specs=[pl.BlockSpec((LATENT, S), lambda p, i: (0, 0))],
        out_specs=pl.BlockSpec((1, RB_CODES, S), lambda p, i: (9 + p, i, 0)),
        out_shape=jax.ShapeDtypeStruct((C_OUT, N_RAYS, S), jnp.float32),
    )(codes_nt)


def _pts_planes_body(prev_ref, t_ref, rays_ref, out_ref, depth_ref):
    del prev_ref
    t = t_ref[...].T
    rays = rays_ref[...]
    for c in range(3):
        o_c = rays[:, c:c + 1]
        d_c = rays[:, 3 + c:4 + c]
        pts = o_c + t * d_c
        out_ref[c] = pts
        out_ref[c + 6] = pts
        out_ref[c + 3] = jnp.broadcast_to(d_c, (RB_C, S))
    d = rays[:, 3:6]
    dd = jnp.sum(d * d, axis=1, keepdims=True)
    depth_ref[...] = jnp.abs(t) * jnp.sqrt(dd)


def _pts_planes(out_partial, t_sorted, rays):
    grid = N_RAYS // RB_C
    return pl.pallas_call(
        _pts_planes_body,
        grid=(grid,),
        in_specs=[
            pl.BlockSpec((1, 8, S), lambda i: (0, 0, 0)),
            pl.BlockSpec((S, RB_C), lambda i: (0, i)),
            pl.BlockSpec((RB_C, 8), lambda i: (i, 0)),
        ],
        out_specs=[
            pl.BlockSpec((9, RB_C, S), lambda i: (0, i, 0)),
            pl.BlockSpec((RB_C, S), lambda i: (i, 0)),
        ],
        out_shape=[
            jax.ShapeDtypeStruct((C_OUT, N_RAYS, S), jnp.float32),
            jax.ShapeDtypeStruct((N_RAYS, S), jnp.float32),
        ],
        input_output_aliases={0: 0},
    )(out_partial, t_sorted, rays)


def kernel(rays, radii, latent_codes):
    t, k3 = _compute_tk(rays.T, radii)
    k = k3.reshape(N_RAYS)
    codes_nt = _compute_codes(latent_codes.T)
    t_sorted = _sc_sort(t, k)
    out_partial = _codes_planes(codes_nt)
    out25, depth = _pts_planes(out_partial, t_sorted, rays)
    return jnp.transpose(out25, (1, 2, 0)), depth


# speedup vs baseline: 1.2281x; 1.2281x over previous
import functools
import jax
import jax.numpy as jnp
from jax import lax
from jax.experimental import pallas as pl
from jax.experimental.pallas import tpu as pltpu
from jax.experimental.pallas import tpu_sc as plsc

N_RAYS = 16384
S = 128
C_OUT = 25
LATENT = 16

RB_A = 256
RB_C = 128
SC_WORKERS = 32
SC_CHUNK = N_RAYS // SC_WORKERS
SC_SUB = 256
GROUPS = SC_SUB // 16
ILV = 4


def _tk_body(rays_t_ref, radii_ref, t_ref, k_ref):
    rt = rays_t_ref[...]
    o = rt[0:3, :]
    d = rt[3:6, :]
    od = jnp.sum(o * d, axis=0, keepdims=True)
    dd = jnp.sum(d * d, axis=0, keepdims=True)
    oo = jnp.sum(o * o, axis=0, keepdims=True)
    r = radii_ref[...]
    disc = od * od - dd * (oo - r * r)
    disc = jnp.maximum(disc, 0.0)
    t = (-od + jnp.sqrt(disc)) / dd
    t_ref[...] = t
    k = jnp.sum((t > 0.0).astype(jnp.int32), axis=0)
    k_ref[...] = k.reshape(1, 1, RB_A)


def _compute_tk(rays_t, radii):
    grid = N_RAYS // RB_A
    return pl.pallas_call(
        _tk_body,
        grid=(grid,),
        in_specs=[
            pl.BlockSpec((8, RB_A), lambda i: (0, i)),
            pl.BlockSpec((S, 1), lambda i: (0, 0)),
        ],
        out_specs=[
            pl.BlockSpec((S, RB_A), lambda i: (0, i)),
            pl.BlockSpec((1, 1, RB_A), lambda i: (i, 0, 0)),
        ],
        out_shape=[
            jax.ShapeDtypeStruct((S, N_RAYS), jnp.float32),
            jax.ShapeDtypeStruct((grid, 1, RB_A), jnp.int32),
        ],
    )(rays_t, radii.reshape(S, 1))


def _codes_body(ct_ref, out_ref):
    ct = ct_ref[...]
    nrm = jnp.sqrt(jnp.sum(ct * ct, axis=0, keepdims=True))
    out_ref[...] = ct / jnp.maximum(nrm, 1e-12)


def _compute_codes(codes_t):
    return pl.pallas_call(
        _codes_body,
        out_shape=jax.ShapeDtypeStruct((LATENT, S), jnp.float32),
    )(codes_t)


def _sc_merge_body(t_hbm, k_hbm, out_hbm, t_v, o_v, k_v):
    wid = lax.axis_index("s") * 2 + lax.axis_index("c")
    big = jnp.float32(3e38)
    lanes = jnp.arange(16, dtype=jnp.int32)
    for sub in range(SC_CHUNK // SC_SUB):
        base = wid * SC_CHUNK + sub * SC_SUB
        pltpu.sync_copy(t_hbm.at[:, pl.ds(base, SC_SUB)], t_v)
        pltpu.sync_copy(k_hbm.at[pl.ds(base, SC_SUB)], k_v)
        for gb in range(GROUPS // ILV):
            rids = []
            los = []
            his = []
            for u in range(ILV):
                g = gb * ILV + u
                rids.append(g * 16 + lanes)
                kv = k_v[pl.ds(g * 16, 16)]
                los.append(kv - 1)
                his.append(kv)

            def step(j, carry):
                lohi = list(carry)
                jcol = jnp.full((16,), 0, jnp.int32) + j
                out = []
                for u in range(ILV):
                    lo, hi = lohi[2 * u], lohi[2 * u + 1]
                    vlo = lo >= 0
                    vhi = hi < S
                    ilo = jnp.clip(lo, 0, S - 1)
                    ihi = jnp.clip(hi, 0, S - 1)
                    tl = plsc.load_gather(t_v, [ilo, rids[u]])
                    th = plsc.load_gather(t_v, [ihi, rids[u]])
                    al = jnp.where(vlo, jnp.abs(tl), big)
                    ah = jnp.where(vhi, jnp.abs(th), big)
                    take = al <= ah
                    chosen = jnp.where(take, tl, th)
                    plsc.store_scatter(o_v, [jcol, rids[u]], chosen)
                    out.append(jnp.where(take, lo - 1, lo))
                    out.append(jnp.where(take, hi, hi + 1))
                return tuple(out)

            init = []
            for u in range(ILV):
                init.append(los[u])
                init.append(his[u])
            lax.fori_loop(0, S, step, tuple(init))
        pltpu.sync_copy(o_v, out_hbm.at[:, pl.ds(base, SC_SUB)])


def _sc_sort(t, k):
    mesh = plsc.VectorSubcoreMesh(core_axis_name="c", subcore_axis_name="s")
    fn = functools.partial(
        pl.kernel,
        mesh=mesh,
        out_type=jax.ShapeDtypeStruct((S, N_RAYS), jnp.float32),
        scratch_types=[
            pltpu.VMEM((S, SC_SUB), jnp.float32),
            pltpu.VMEM((S, SC_SUB), jnp.float32),
            pltpu.VMEM((SC_SUB,), jnp.int32),
        ],
        compiler_params=pltpu.CompilerParams(needs_layout_passes=False),
    )(_sc_merge_body)
    return fn(t, k)


def _assemble_body(t_ref, rays_ref, ct_ref, out_ref, depth_ref):
    t = t_ref[...].T
    rays = rays_ref[...]
    for c in range(3):
        o_c = rays[:, c:c + 1]
        d_c = rays[:, 3 + c:4 + c]
        pts = o_c + t * d_c
        out_ref[c] = pts
        out_ref[c + 6] = pts
        out_ref[c + 3] = jnp.broadcast_to(d_c, (RB_C, S))
    for q in range(LATENT):
        out_ref[9 + q] = jnp.broadcast_to(ct_ref[q:q + 1, :], (RB_C, S))
    d = rays[:, 3:6]
    dd = jnp.sum(d * d, axis=1, keepdims=True)
    depth_ref[...] = jnp.abs(t) * jnp.sqrt(dd)


def _assemble(t_sorted, rays, codes_nt):
    grid = N_RAYS // RB_C
    return pl.pallas_call(
        _assemble_body,
        grid=(grid,),
        in_specs=[
            pl.BlockSpec((S, RB_C), lambda i: (0, i)),
            pl.BlockSpec((RB_C, 8), lambda i: (i, 0)),
            pl.BlockSpec((LATENT, S), lambda i: (0, 0)),
        ],
        out_specs=[
            pl.BlockSpec((C_OUT, RB_C, S), lambda i: (0, i, 0)),
            pl.BlockSpec((RB_C, S), lambda i: (i, 0)),
        ],
        out_shape=[
            jax.ShapeDtypeStruct((C_OUT, N_RAYS, S), jnp.float32),
            jax.ShapeDtypeStruct((N_RAYS, S), jnp.float32),
        ],
    )(t_sorted, rays, codes_nt)


def kernel(rays, radii, latent_codes):
    t, k3 = _compute_tk(rays.T, radii)
    k = k3.reshape(N_RAYS)
    codes_nt = _compute_codes(latent_codes.T)
    t_sorted = _sc_sort(t, k)
    out25, depth = _assemble(t_sorted, rays, codes_nt)
    return jnp.transpose(out25, (1, 2, 0)), depth


# speedup vs baseline: 1.5846x; 1.2903x over previous
import functools
import jax
import jax.numpy as jnp
from jax import lax
from jax.experimental import pallas as pl
from jax.experimental.pallas import tpu as pltpu
from jax.experimental.pallas import tpu_sc as plsc

N_RAYS = 16384
S = 128
C_OUT = 25
LATENT = 16

RB_A = 512
RB_C = 256
SC_WORKERS = 32
SC_CHUNK = N_RAYS // SC_WORKERS
SC_SUB = 256
GROUPS = SC_SUB // 16
ILV = 4


def _tk_body(rays_t_ref, radii_ref, t_ref, k_ref):
    rt = rays_t_ref[...]
    o = rt[0:3, :]
    d = rt[3:6, :]
    od = jnp.sum(o * d, axis=0, keepdims=True)
    dd = jnp.sum(d * d, axis=0, keepdims=True)
    oo = jnp.sum(o * o, axis=0, keepdims=True)
    r = radii_ref[...]
    disc = od * od - dd * (oo - r * r)
    disc = jnp.maximum(disc, 0.0)
    t = (-od + jnp.sqrt(disc)) / dd
    t_ref[...] = t
    k = jnp.sum((t > 0.0).astype(jnp.int32), axis=0)
    k_ref[...] = k.reshape(1, 1, RB_A)


def _compute_tk(rays_t, radii):
    grid = N_RAYS // RB_A
    return pl.pallas_call(
        _tk_body,
        grid=(grid,),
        in_specs=[
            pl.BlockSpec((8, RB_A), lambda i: (0, i)),
            pl.BlockSpec((S, 1), lambda i: (0, 0)),
        ],
        out_specs=[
            pl.BlockSpec((S, RB_A), lambda i: (0, i)),
            pl.BlockSpec((1, 1, RB_A), lambda i: (i, 0, 0)),
        ],
        out_shape=[
            jax.ShapeDtypeStruct((S, N_RAYS), jnp.float32),
            jax.ShapeDtypeStruct((grid, 1, RB_A), jnp.int32),
        ],
    )(rays_t, radii.reshape(S, 1))


def _codes_body(ct_ref, out_ref):
    ct = ct_ref[...]
    nrm = jnp.sqrt(jnp.sum(ct * ct, axis=0, keepdims=True))
    out_ref[...] = ct / jnp.maximum(nrm, 1e-12)


def _compute_codes(codes_t):
    return pl.pallas_call(
        _codes_body,
        out_shape=jax.ShapeDtypeStruct((LATENT, S), jnp.float32),
    )(codes_t)


def _sc_merge_body(t_hbm, k_hbm, out_hbm, t_v, o_v, k_v):
    wid = lax.axis_index("s") * 2 + lax.axis_index("c")
    big = jnp.float32(3e38)
    lanes = jnp.arange(16, dtype=jnp.int32)
    for sub in range(SC_CHUNK // SC_SUB):
        base = wid * SC_CHUNK + sub * SC_SUB
        pltpu.sync_copy(t_hbm.at[:, pl.ds(base, SC_SUB)], t_v)
        pltpu.sync_copy(k_hbm.at[pl.ds(base, SC_SUB)], k_v)
        for gb in range(GROUPS // ILV):
            rids = []
            los = []
            his = []
            for u in range(ILV):
                g = gb * ILV + u
                rids.append(g * 16 + lanes)
                kv = k_v[pl.ds(g * 16, 16)]
                los.append(kv - 1)
                his.append(kv)

            def step(j, carry):
                lohi = list(carry)
                jcol = jnp.full((16,), 0, jnp.int32) + j
                out = []
                for u in range(ILV):
                    lo, hi = lohi[2 * u], lohi[2 * u + 1]
                    vlo = lo >= 0
                    vhi = hi < S
                    ilo = jnp.clip(lo, 0, S - 1)
                    ihi = jnp.clip(hi, 0, S - 1)
                    tl = plsc.load_gather(t_v, [ilo, rids[u]])
                    th = plsc.load_gather(t_v, [ihi, rids[u]])
                    al = jnp.where(vlo, jnp.abs(tl), big)
                    ah = jnp.where(vhi, jnp.abs(th), big)
                    take = al <= ah
                    chosen = jnp.where(take, tl, th)
                    plsc.store_scatter(o_v, [jcol, rids[u]], chosen)
                    out.append(jnp.where(take, lo - 1, lo))
                    out.append(jnp.where(take, hi, hi + 1))
                return tuple(out)

            init = []
            for u in range(ILV):
                init.append(los[u])
                init.append(his[u])
            lax.fori_loop(0, S, step, tuple(init))
        pltpu.sync_copy(o_v, out_hbm.at[:, pl.ds(base, SC_SUB)])


def _sc_sort(t, k):
    mesh = plsc.VectorSubcoreMesh(core_axis_name="c", subcore_axis_name="s")
    fn = functools.partial(
        pl.kernel,
        mesh=mesh,
        out_type=jax.ShapeDtypeStruct((S, N_RAYS), jnp.float32),
        scratch_types=[
            pltpu.VMEM((S, SC_SUB), jnp.float32),
            pltpu.VMEM((S, SC_SUB), jnp.float32),
            pltpu.VMEM((SC_SUB,), jnp.int32),
        ],
        compiler_params=pltpu.CompilerParams(needs_layout_passes=False),
    )(_sc_merge_body)
    return fn(t, k)


def _assemble_body(t_ref, rays_ref, ct_ref, out_ref, depth_ref):
    t = t_ref[...].T
    rays = rays_ref[...]
    for c in range(3):
        o_c = rays[:, c:c + 1]
        d_c = rays[:, 3 + c:4 + c]
        pts = o_c + t * d_c
        out_ref[c] = pts
        out_ref[c + 6] = pts
        out_ref[c + 3] = jnp.broadcast_to(d_c, (RB_C, S))
    for q in range(LATENT):
        out_ref[9 + q] = jnp.broadcast_to(ct_ref[q:q + 1, :], (RB_C, S))
    d = rays[:, 3:6]
    dd = jnp.sum(d * d, axis=1, keepdims=True)
    depth_ref[...] = jnp.abs(t) * jnp.sqrt(dd)


def _assemble(t_sorted, rays, codes_nt):
    grid = N_RAYS // RB_C
    return pl.pallas_call(
        _assemble_body,
        grid=(grid,),
        in_specs=[
            pl.BlockSpec((S, RB_C), lambda i: (0, i)),
            pl.BlockSpec((RB_C, 8), lambda i: (i, 0)),
            pl.BlockSpec((LATENT, S), lambda i: (0, 0)),
        ],
        out_specs=[
            pl.BlockSpec((C_OUT, RB_C, S), lambda i: (0, i, 0)),
            pl.BlockSpec((RB_C, S), lambda i: (i, 0)),
        ],
        out_shape=[
            jax.ShapeDtypeStruct((C_OUT, N_RAYS, S), jnp.float32),
            jax.ShapeDtypeStruct((N_RAYS, S), jnp.float32),
        ],
    )(t_sorted, rays, codes_nt)


def kernel(rays, radii, latent_codes):
    t, k3 = _compute_tk(rays.T, radii)
    k = k3.reshape(N_RAYS)
    codes_nt = _compute_codes(latent_codes.T)
    t_sorted = _sc_sort(t, k)
    out25, depth = _assemble(t_sorted, rays, codes_nt)
    return jnp.transpose(out25, (1, 2, 0)), depth


# speedup vs baseline: 1.8460x; 1.1650x over previous
import functools
import jax
import jax.numpy as jnp
from jax import lax
from jax.experimental import pallas as pl
from jax.experimental.pallas import tpu as pltpu
from jax.experimental.pallas import tpu_sc as plsc

N_RAYS = 16384
S = 128
C_OUT = 25
LATENT = 16

RB_A = 1024
RB_C = 512
SC_WORKERS = 32
SC_CHUNK = N_RAYS // SC_WORKERS
SC_SUB = 256
GROUPS = SC_SUB // 16
ILV = 4


def _tk_body(rays_t_ref, radii_ref, t_ref, k_ref):
    rt = rays_t_ref[...]
    o = rt[0:3, :]
    d = rt[3:6, :]
    od = jnp.sum(o * d, axis=0, keepdims=True)
    dd = jnp.sum(d * d, axis=0, keepdims=True)
    oo = jnp.sum(o * o, axis=0, keepdims=True)
    r = radii_ref[...]
    disc = od * od - dd * (oo - r * r)
    disc = jnp.maximum(disc, 0.0)
    t = (-od + jnp.sqrt(disc)) / dd
    t_ref[...] = t
    k = jnp.sum((t > 0.0).astype(jnp.int32), axis=0)
    k_ref[...] = k.reshape(1, 1, RB_A)


def _compute_tk(rays_t, radii):
    grid = N_RAYS // RB_A
    return pl.pallas_call(
        _tk_body,
        grid=(grid,),
        in_specs=[
            pl.BlockSpec((8, RB_A), lambda i: (0, i)),
            pl.BlockSpec((S, 1), lambda i: (0, 0)),
        ],
        out_specs=[
            pl.BlockSpec((S, RB_A), lambda i: (0, i)),
            pl.BlockSpec((1, 1, RB_A), lambda i: (i, 0, 0)),
        ],
        out_shape=[
            jax.ShapeDtypeStruct((S, N_RAYS), jnp.float32),
            jax.ShapeDtypeStruct((grid, 1, RB_A), jnp.int32),
        ],
    )(rays_t, radii.reshape(S, 1))


def _codes_body(ct_ref, out_ref):
    ct = ct_ref[...]
    nrm = jnp.sqrt(jnp.sum(ct * ct, axis=0, keepdims=True))
    out_ref[...] = ct / jnp.maximum(nrm, 1e-12)


def _compute_codes(codes_t):
    return pl.pallas_call(
        _codes_body,
        out_shape=jax.ShapeDtypeStruct((LATENT, S), jnp.float32),
    )(codes_t)


def _sc_merge_body(t_hbm, k_hbm, out_hbm, t_v, o_v, k_v):
    wid = lax.axis_index("s") * 2 + lax.axis_index("c")
    big = jnp.float32(3e38)
    lanes = jnp.arange(16, dtype=jnp.int32)
    for sub in range(SC_CHUNK // SC_SUB):
        base = wid * SC_CHUNK + sub * SC_SUB
        pltpu.sync_copy(t_hbm.at[:, pl.ds(base, SC_SUB)], t_v)
        pltpu.sync_copy(k_hbm.at[pl.ds(base, SC_SUB)], k_v)
        for gb in range(GROUPS // ILV):
            rids = []
            los = []
            his = []
            for u in range(ILV):
                g = gb * ILV + u
                rids.append(g * 16 + lanes)
                kv = k_v[pl.ds(g * 16, 16)]
                los.append(kv - 1)
                his.append(kv)

            def step(j, carry):
                lohi = list(carry)
                jcol = jnp.full((16,), 0, jnp.int32) + j
                out = []
                for u in range(ILV):
                    lo, hi = lohi[2 * u], lohi[2 * u + 1]
                    vlo = lo >= 0
                    vhi = hi < S
                    ilo = jnp.clip(lo, 0, S - 1)
                    ihi = jnp.clip(hi, 0, S - 1)
                    tl = plsc.load_gather(t_v, [ilo, rids[u]])
                    th = plsc.load_gather(t_v, [ihi, rids[u]])
                    al = jnp.where(vlo, jnp.abs(tl), big)
                    ah = jnp.where(vhi, jnp.abs(th), big)
                    take = al <= ah
                    chosen = jnp.where(take, tl, th)
                    plsc.store_scatter(o_v, [jcol, rids[u]], chosen)
                    out.append(jnp.where(take, lo - 1, lo))
                    out.append(jnp.where(take, hi, hi + 1))
                return tuple(out)

            init = []
            for u in range(ILV):
                init.append(los[u])
                init.append(his[u])
            lax.fori_loop(0, S, step, tuple(init))
        pltpu.sync_copy(o_v, out_hbm.at[:, pl.ds(base, SC_SUB)])


def _sc_sort(t, k):
    mesh = plsc.VectorSubcoreMesh(core_axis_name="c", subcore_axis_name="s")
    fn = functools.partial(
        pl.kernel,
        mesh=mesh,
        out_type=jax.ShapeDtypeStruct((S, N_RAYS), jnp.float32),
        scratch_types=[
            pltpu.VMEM((S, SC_SUB), jnp.float32),
            pltpu.VMEM((S, SC_SUB), jnp.float32),
            pltpu.VMEM((SC_SUB,), jnp.int32),
        ],
        compiler_params=pltpu.CompilerParams(needs_layout_passes=False),
    )(_sc_merge_body)
    return fn(t, k)


def _assemble_body(t_ref, rays_ref, ct_ref, out_ref, depth_ref):
    t = t_ref[...].T
    rays = rays_ref[...]
    for c in range(3):
        o_c = rays[:, c:c + 1]
        d_c = rays[:, 3 + c:4 + c]
        pts = o_c + t * d_c
        out_ref[c] = pts
        out_ref[c + 6] = pts
        out_ref[c + 3] = jnp.broadcast_to(d_c, (RB_C, S))
    for q in range(LATENT):
        out_ref[9 + q] = jnp.broadcast_to(ct_ref[q:q + 1, :], (RB_C, S))
    d = rays[:, 3:6]
    dd = jnp.sum(d * d, axis=1, keepdims=True)
    depth_ref[...] = jnp.abs(t) * jnp.sqrt(dd)


def _assemble(t_sorted, rays, codes_nt):
    grid = N_RAYS // RB_C
    return pl.pallas_call(
        _assemble_body,
        grid=(grid,),
        in_specs=[
            pl.BlockSpec((S, RB_C), lambda i: (0, i)),
            pl.BlockSpec((RB_C, 8), lambda i: (i, 0)),
            pl.BlockSpec((LATENT, S), lambda i: (0, 0)),
        ],
        out_specs=[
            pl.BlockSpec((C_OUT, RB_C, S), lambda i: (0, i, 0)),
            pl.BlockSpec((RB_C, S), lambda i: (i, 0)),
        ],
        out_shape=[
            jax.ShapeDtypeStruct((C_OUT, N_RAYS, S), jnp.float32),
            jax.ShapeDtypeStruct((N_RAYS, S), jnp.float32),
        ],
    )(t_sorted, rays, codes_nt)


def kernel(rays, radii, latent_codes):
    t, k3 = _compute_tk(rays.T, radii)
    k = k3.reshape(N_RAYS)
    codes_nt = _compute_codes(latent_codes.T)
    t_sorted = _sc_sort(t, k)
    out25, depth = _assemble(t_sorted, rays, codes_nt)
    return jnp.transpose(out25, (1, 2, 0)), depth


# speedup vs baseline: 1.9065x; 1.0328x over previous
import functools
import jax
import jax.numpy as jnp
from jax import lax
from jax.experimental import pallas as pl
from jax.experimental.pallas import tpu as pltpu
from jax.experimental.pallas import tpu_sc as plsc

N_RAYS = 16384
S = 128
C_OUT = 25
LATENT = 16

RB_A = 2048
RB_C = 1024
SC_WORKERS = 32
SC_CHUNK = N_RAYS // SC_WORKERS
SC_SUB = 256
GROUPS = SC_SUB // 16
ILV = 4


def _tk_body(rays_t_ref, radii_ref, t_ref, k_ref):
    rt = rays_t_ref[...]
    o = rt[0:3, :]
    d = rt[3:6, :]
    od = jnp.sum(o * d, axis=0, keepdims=True)
    dd = jnp.sum(d * d, axis=0, keepdims=True)
    oo = jnp.sum(o * o, axis=0, keepdims=True)
    r = radii_ref[...]
    disc = od * od - dd * (oo - r * r)
    disc = jnp.maximum(disc, 0.0)
    t = (-od + jnp.sqrt(disc)) / dd
    t_ref[...] = t
    k = jnp.sum((t > 0.0).astype(jnp.int32), axis=0)
    k_ref[...] = k.reshape(1, 1, RB_A)


def _compute_tk(rays_t, radii):
    grid = N_RAYS // RB_A
    return pl.pallas_call(
        _tk_body,
        grid=(grid,),
        in_specs=[
            pl.BlockSpec((8, RB_A), lambda i: (0, i)),
            pl.BlockSpec((S, 1), lambda i: (0, 0)),
        ],
        out_specs=[
            pl.BlockSpec((S, RB_A), lambda i: (0, i)),
            pl.BlockSpec((1, 1, RB_A), lambda i: (i, 0, 0)),
        ],
        out_shape=[
            jax.ShapeDtypeStruct((S, N_RAYS), jnp.float32),
            jax.ShapeDtypeStruct((grid, 1, RB_A), jnp.int32),
        ],
    )(rays_t, radii.reshape(S, 1))


def _codes_body(ct_ref, out_ref):
    ct = ct_ref[...]
    nrm = jnp.sqrt(jnp.sum(ct * ct, axis=0, keepdims=True))
    out_ref[...] = ct / jnp.maximum(nrm, 1e-12)


def _compute_codes(codes_t):
    return pl.pallas_call(
        _codes_body,
        out_shape=jax.ShapeDtypeStruct((LATENT, S), jnp.float32),
    )(codes_t)


def _sc_merge_body(t_hbm, k_hbm, out_hbm, t_v, o_v, k_v):
    wid = lax.axis_index("s") * 2 + lax.axis_index("c")
    big = jnp.float32(3e38)
    lanes = jnp.arange(16, dtype=jnp.int32)
    for sub in range(SC_CHUNK // SC_SUB):
        base = wid * SC_CHUNK + sub * SC_SUB
        pltpu.sync_copy(t_hbm.at[:, pl.ds(base, SC_SUB)], t_v)
        pltpu.sync_copy(k_hbm.at[pl.ds(base, SC_SUB)], k_v)
        for gb in range(GROUPS // ILV):
            rids = []
            los = []
            his = []
            for u in range(ILV):
                g = gb * ILV + u
                rids.append(g * 16 + lanes)
                kv = k_v[pl.ds(g * 16, 16)]
                los.append(kv - 1)
                his.append(kv)

            def step(j, carry):
                lohi = list(carry)
                jcol = jnp.full((16,), 0, jnp.int32) + j
                out = []
                for u in range(ILV):
                    lo, hi = lohi[2 * u], lohi[2 * u + 1]
                    vlo = lo >= 0
                    vhi = hi < S
                    ilo = jnp.clip(lo, 0, S - 1)
                    ihi = jnp.clip(hi, 0, S - 1)
                    tl = plsc.load_gather(t_v, [ilo, rids[u]])
                    th = plsc.load_gather(t_v, [ihi, rids[u]])
                    al = jnp.where(vlo, jnp.abs(tl), big)
                    ah = jnp.where(vhi, jnp.abs(th), big)
                    take = al <= ah
                    chosen = jnp.where(take, tl, th)
                    plsc.store_scatter(o_v, [jcol, rids[u]], chosen)
                    out.append(jnp.where(take, lo - 1, lo))
                    out.append(jnp.where(take, hi, hi + 1))
                return tuple(out)

            init = []
            for u in range(ILV):
                init.append(los[u])
                init.append(his[u])
            lax.fori_loop(0, S, step, tuple(init))
        pltpu.sync_copy(o_v, out_hbm.at[:, pl.ds(base, SC_SUB)])


def _sc_sort(t, k):
    mesh = plsc.VectorSubcoreMesh(core_axis_name="c", subcore_axis_name="s")
    fn = functools.partial(
        pl.kernel,
        mesh=mesh,
        out_type=jax.ShapeDtypeStruct((S, N_RAYS), jnp.float32),
        scratch_types=[
            pltpu.VMEM((S, SC_SUB), jnp.float32),
            pltpu.VMEM((S, SC_SUB), jnp.float32),
            pltpu.VMEM((SC_SUB,), jnp.int32),
        ],
        compiler_params=pltpu.CompilerParams(needs_layout_passes=False),
    )(_sc_merge_body)
    return fn(t, k)


def _assemble_body(t_ref, rays_ref, ct_ref, out_ref, depth_ref):
    t = t_ref[...].T
    rays = rays_ref[...]
    for c in range(3):
        o_c = rays[:, c:c + 1]
        d_c = rays[:, 3 + c:4 + c]
        pts = o_c + t * d_c
        out_ref[c] = pts
        out_ref[c + 6] = pts
        out_ref[c + 3] = jnp.broadcast_to(d_c, (RB_C, S))
    for q in range(LATENT):
        out_ref[9 + q] = jnp.broadcast_to(ct_ref[q:q + 1, :], (RB_C, S))
    d = rays[:, 3:6]
    dd = jnp.sum(d * d, axis=1, keepdims=True)
    depth_ref[...] = jnp.abs(t) * jnp.sqrt(dd)


def _assemble(t_sorted, rays, codes_nt):
    grid = N_RAYS // RB_C
    return pl.pallas_call(
        _assemble_body,
        grid=(grid,),
        in_specs=[
            pl.BlockSpec((S, RB_C), lambda i: (0, i)),
            pl.BlockSpec((RB_C, 8), lambda i: (i, 0)),
            pl.BlockSpec((LATENT, S), lambda i: (0, 0)),
        ],
        out_specs=[
            pl.BlockSpec((C_OUT, RB_C, S), lambda i: (0, i, 0)),
            pl.BlockSpec((RB_C, S), lambda i: (i, 0)),
        ],
        out_shape=[
            jax.ShapeDtypeStruct((C_OUT, N_RAYS, S), jnp.float32),
            jax.ShapeDtypeStruct((N_RAYS, S), jnp.float32),
        ],
    )(t_sorted, rays, codes_nt)


def kernel(rays, radii, latent_codes):
    t, k3 = _compute_tk(rays.T, radii)
    k = k3.reshape(N_RAYS)
    codes_nt = _compute_codes(latent_codes.T)
    t_sorted = _sc_sort(t, k)
    out25, depth = _assemble(t_sorted, rays, codes_nt)
    return jnp.transpose(out25, (1, 2, 0)), depth
